# baseline (device time: 19795 ns/iter reference)
import jax
import jax.numpy as jnp
from jax import lax
from jax.experimental import pallas as pl
from jax.experimental.pallas import tpu as pltpu

N_DEV = 16
C = 4
R = 4


def kernel(x):
    m_per, n = x.shape
    assert m_per % R == 0 and n % C == 0
    m_blk = m_per // R
    n_blk = n // C

    def body(x_ref, out_ref, acc_ref, send_buf, recv_buf, send_sems, recv_sems):
        c = pl.program_id(0)
        r = pl.program_id(1)
        my_pos = lax.axis_index("i")

        blk_max = jnp.max(x_ref[...], axis=0, keepdims=True)

        @pl.when(jnp.logical_and(c == 0, r == 0))
        def _():
            barrier_sem = pltpu.get_barrier_semaphore()
            for off in range(1, N_DEV):
                pl.semaphore_signal(
                    barrier_sem, inc=1,
                    device_id=((my_pos + off) % N_DEV,),
                    device_id_type=pl.DeviceIdType.MESH,
                )

        @pl.when(r == 0)
        def _():
            acc_ref[...] = blk_max

        @pl.when(r > 0)
        def _():
            acc_ref[...] = jnp.maximum(acc_ref[...], blk_max)

        def chunk_rdma(cc, off):
            return pltpu.make_async_remote_copy(
                src_ref=send_buf.at[cc],
                dst_ref=recv_buf.at[cc, off - 1],
                send_sem=send_sems.at[cc, off - 1],
                recv_sem=recv_sems.at[cc, off - 1],
                device_id=((my_pos + off) % N_DEV,),
                device_id_type=pl.DeviceIdType.MESH,
            )

        def broadcast_chunk(cc):
            for off in range(1, N_DEV):
                chunk_rdma(cc, off).start()

        for cc in range(C):
            @pl.when(jnp.logical_and(c == cc, r == R - 1))
            def _(cc=cc):
                send_buf[cc] = acc_ref[...]
                if cc == 1:
                    pl.semaphore_wait(pltpu.get_barrier_semaphore(), N_DEV - 1)
                    broadcast_chunk(0)
                if cc >= 1:
                    broadcast_chunk(cc)
                if cc == C - 1:
                    for fc in range(C):
                        for off in range(1, N_DEV):
                            chunk_rdma(fc, off).wait_recv()
                        folded = jnp.maximum(
                            send_buf[fc],
                            jnp.max(recv_buf[fc], axis=0),
                        )
                        out_ref[:, fc * n_blk:(fc + 1) * n_blk] = folded
                    for fc in range(C):
                        for off in range(1, N_DEV):
                            chunk_rdma(fc, off).wait_send()

    return pl.pallas_call(
        body,
        grid=(C, R),
        out_shape=jax.ShapeDtypeStruct((1, n), x.dtype),
        in_specs=[pl.BlockSpec((m_blk, n_blk), lambda c, r: (r, c))],
        out_specs=pl.BlockSpec((1, n), lambda c, r: (0, 0)),
        scratch_shapes=[
            pltpu.VMEM((1, n_blk), x.dtype),
            pltpu.VMEM((C, 1, n_blk), x.dtype),
            pltpu.VMEM((C, N_DEV - 1, 1, n_blk), x.dtype),
            pltpu.SemaphoreType.DMA((C, N_DEV - 1)),
            pltpu.SemaphoreType.DMA((C, N_DEV - 1)),
        ],
        compiler_params=pltpu.CompilerParams(collective_id=0),
    )(x)
